# baseline (speedup 1.0000x reference)
import jax
import jax.numpy as jnp
from jax.experimental import pallas as pl
from jax.experimental.pallas import tpu as pltpu


def _pd_kernel(q_ref, ctx_ref, msk_ref, wtop_ref, wbot_ref, wsin_ref,
               wsout_ref, bsw_ref, out_ref, wgt_ref, sw_ref):
  q32 = q_ref[0]
  qb = q32.astype(jnp.bfloat16)
  cb = ctx_ref[0].astype(jnp.bfloat16)

  scores = jax.lax.dot_general(qb, cb, (((1,), (1,)), ((), ())),
                               preferred_element_type=jnp.float32)
  scores = jnp.where(msk_ref[0] != 0.0, scores, jnp.float32(-1e30))

  m = jnp.max(scores, axis=-1, keepdims=True)
  p = jnp.exp(scores - m)
  l = jnp.sum(p, axis=-1, keepdims=True)
  inv_l = pl.reciprocal(l, approx=False)
  wgt_ref[0] = p * inv_l

  attn = jax.lax.dot_general(p.astype(jnp.bfloat16), cb,
                             (((1,), (0,)), ((), ())),
                             preferred_element_type=jnp.float32) * inv_l

  h = (jnp.dot(qb, wtop_ref[...], preferred_element_type=jnp.float32) +
       jnp.dot(attn.astype(jnp.bfloat16), wbot_ref[...],
               preferred_element_type=jnp.float32))
  out = jnp.tanh(h)
  out_ref[0] = out.astype(out_ref.dtype)

  z = (jnp.sum(q32 * wsin_ref[...], axis=-1, keepdims=True) +
       jnp.sum(out * wsout_ref[...], axis=-1, keepdims=True) +
       bsw_ref[...])
  sw = jax.nn.sigmoid(z)
  sw_ref[0] = jnp.broadcast_to(sw, sw_ref.shape[1:]).astype(sw_ref.dtype)


def _round_up(x, m):
  return ((x + m - 1) // m) * m


def kernel(inp, context, atten_mask, w_attn, w_switch, b_switch):
  B, T, D = inp.shape
  _, S, _ = context.shape

  w_top = jnp.asarray(w_attn[:, :D].T, jnp.bfloat16)
  w_bot = jnp.asarray(w_attn[:, D:].T, jnp.bfloat16)
  w_sw_in = jnp.asarray(w_switch[:, :D], jnp.float32)
  w_sw_out = jnp.asarray(w_switch[:, D:], jnp.float32)
  b_sw = jnp.asarray(b_switch, jnp.float32).reshape(1, 1)

  tm = min(128, _round_up(T, 8))
  t_pad, s_pad = _round_up(T, tm), _round_up(S, 128)
  nt = t_pad // tm

  q = jnp.asarray(inp, jnp.float32)
  ctx = jnp.asarray(context, jnp.float32)
  msk = jnp.asarray(atten_mask, jnp.float32)
  if t_pad != T:
    q = jnp.pad(q, ((0, 0), (0, t_pad - T), (0, 0)))
    msk = jnp.pad(msk, ((0, 0), (0, t_pad - T), (0, 0)))
  if s_pad != S:
    ctx = jnp.pad(ctx, ((0, 0), (0, s_pad - S), (0, 0)))
    msk = jnp.pad(msk, ((0, 0), (0, 0), (0, s_pad - S)))

  out_shapes = (
      jax.ShapeDtypeStruct((B, t_pad, D), jnp.float32),
      jax.ShapeDtypeStruct((B, t_pad, s_pad), jnp.float32),
      jax.ShapeDtypeStruct((B, t_pad, 128), jnp.float32),
  )

  out, weight, switch = pl.pallas_call(
      _pd_kernel,
      out_shape=out_shapes,
      grid=(B, nt),
      in_specs=[
          pl.BlockSpec((1, tm, D), lambda b, i: (b, i, 0)),
          pl.BlockSpec((1, s_pad, D), lambda b, i: (b, 0, 0)),
          pl.BlockSpec((1, tm, s_pad), lambda b, i: (b, i, 0)),
          pl.BlockSpec((D, D), lambda b, i: (0, 0)),
          pl.BlockSpec((D, D), lambda b, i: (0, 0)),
          pl.BlockSpec((1, D), lambda b, i: (0, 0)),
          pl.BlockSpec((1, D), lambda b, i: (0, 0)),
          pl.BlockSpec((1, 1), lambda b, i: (0, 0)),
      ],
      out_specs=[
          pl.BlockSpec((1, tm, D), lambda b, i: (b, i, 0)),
          pl.BlockSpec((1, tm, s_pad), lambda b, i: (b, i, 0)),
          pl.BlockSpec((1, tm, 128), lambda b, i: (b, i, 0)),
      ],
      compiler_params=pltpu.CompilerParams(
          dimension_semantics=("parallel", "parallel"),
          vmem_limit_bytes=48 << 20),
  )(q, ctx, msk, w_top, w_bot, w_sw_in, w_sw_out, b_sw)

  return out[:, :T, :], weight[:, :T, :S], switch[:, :T, :1]


# speedup vs baseline: 1.6602x; 1.6602x over previous
import jax
import jax.numpy as jnp
from jax.experimental import pallas as pl
from jax.experimental.pallas import tpu as pltpu


def _pd_kernel(q_ref, ctx_ref, msk_ref, wtop_ref, wbot_ref, wsin_ref,
               wsout_ref, bsw_ref, out_ref, wgt_ref, sw_ref):
  q32 = q_ref[0]
  qb = q32.astype(jnp.bfloat16)
  cb = ctx_ref[0].astype(jnp.bfloat16)

  scores = jax.lax.dot_general(qb, cb, (((1,), (1,)), ((), ())),
                               preferred_element_type=jnp.float32)
  scores = jnp.where(msk_ref[0] != 0.0, scores, jnp.float32(-1e30))

  m = jnp.max(scores, axis=-1, keepdims=True)
  p = jnp.exp(scores - m)
  l = jnp.sum(p, axis=-1, keepdims=True)
  inv_l = pl.reciprocal(l, approx=False)
  wgt_ref[0] = p * inv_l

  attn = jax.lax.dot_general(p.astype(jnp.bfloat16), cb,
                             (((1,), (0,)), ((), ())),
                             preferred_element_type=jnp.float32) * inv_l

  h = (jnp.dot(qb, wtop_ref[...], preferred_element_type=jnp.float32) +
       jnp.dot(attn.astype(jnp.bfloat16), wbot_ref[...],
               preferred_element_type=jnp.float32))
  out = jnp.tanh(h)
  out_ref[0] = out.astype(out_ref.dtype)

  z = (jnp.sum(q32 * wsin_ref[...], axis=-1, keepdims=True) +
       jnp.sum(out * wsout_ref[...], axis=-1, keepdims=True) +
       bsw_ref[...])
  sw = jax.nn.sigmoid(z)
  sw_ref[0] = jnp.broadcast_to(sw, sw_ref.shape[1:]).astype(sw_ref.dtype)


def _round_up(x, m):
  return ((x + m - 1) // m) * m


def kernel(inp, context, atten_mask, w_attn, w_switch, b_switch):
  B, T, D = inp.shape
  _, S, _ = context.shape

  w_top = jnp.asarray(w_attn[:, :D].T, jnp.bfloat16)
  w_bot = jnp.asarray(w_attn[:, D:].T, jnp.bfloat16)
  w_sw_in = jnp.asarray(w_switch[:, :D], jnp.float32)
  w_sw_out = jnp.asarray(w_switch[:, D:], jnp.float32)
  b_sw = jnp.asarray(b_switch, jnp.float32).reshape(1, 1)

  tm = min(512, _round_up(T, 8))
  t_pad, s_pad = _round_up(T, tm), _round_up(S, 128)
  nt = t_pad // tm

  q = jnp.asarray(inp, jnp.float32)
  ctx = jnp.asarray(context, jnp.float32)
  msk = jnp.asarray(atten_mask, jnp.float32)
  if t_pad != T:
    q = jnp.pad(q, ((0, 0), (0, t_pad - T), (0, 0)))
    msk = jnp.pad(msk, ((0, 0), (0, t_pad - T), (0, 0)))
  if s_pad != S:
    ctx = jnp.pad(ctx, ((0, 0), (0, s_pad - S), (0, 0)))
    msk = jnp.pad(msk, ((0, 0), (0, 0), (0, s_pad - S)))

  out_shapes = (
      jax.ShapeDtypeStruct((B, t_pad, D), jnp.float32),
      jax.ShapeDtypeStruct((B, t_pad, s_pad), jnp.float32),
      jax.ShapeDtypeStruct((B, t_pad, 128), jnp.float32),
  )

  out, weight, switch = pl.pallas_call(
      _pd_kernel,
      out_shape=out_shapes,
      grid=(B, nt),
      in_specs=[
          pl.BlockSpec((1, tm, D), lambda b, i: (b, i, 0)),
          pl.BlockSpec((1, s_pad, D), lambda b, i: (b, 0, 0)),
          pl.BlockSpec((1, tm, s_pad), lambda b, i: (b, i, 0)),
          pl.BlockSpec((D, D), lambda b, i: (0, 0)),
          pl.BlockSpec((D, D), lambda b, i: (0, 0)),
          pl.BlockSpec((1, D), lambda b, i: (0, 0)),
          pl.BlockSpec((1, D), lambda b, i: (0, 0)),
          pl.BlockSpec((1, 1), lambda b, i: (0, 0)),
      ],
      out_specs=[
          pl.BlockSpec((1, tm, D), lambda b, i: (b, i, 0)),
          pl.BlockSpec((1, tm, s_pad), lambda b, i: (b, i, 0)),
          pl.BlockSpec((1, tm, 128), lambda b, i: (b, i, 0)),
      ],
      compiler_params=pltpu.CompilerParams(
          dimension_semantics=("parallel", "parallel"),
          vmem_limit_bytes=48 << 20),
  )(q, ctx, msk, w_top, w_bot, w_sw_in, w_sw_out, b_sw)

  return out[:, :T, :], weight[:, :T, :S], switch[:, :T, :1]


# speedup vs baseline: 1.8687x; 1.1256x over previous
import jax
import jax.numpy as jnp
from jax.experimental import pallas as pl
from jax.experimental.pallas import tpu as pltpu


def _pd_kernel(q_ref, ctx_ref, msk_ref, wattn_ref, wsw_ref, bsw_ref,
               out_ref, wgt_ref, sw_ref, *, D):
  q32 = q_ref[0]
  qb = q32.astype(jnp.bfloat16)
  cb = ctx_ref[0].astype(jnp.bfloat16)

  scores = jax.lax.dot_general(qb, cb, (((1,), (1,)), ((), ())),
                               preferred_element_type=jnp.float32)
  scores = jnp.where(msk_ref[0] != 0.0, scores, jnp.float32(-1e30))

  m = jnp.max(scores, axis=-1, keepdims=True)
  p = jnp.exp(scores - m)
  l = jnp.sum(p, axis=-1, keepdims=True)
  inv_l = pl.reciprocal(l, approx=False)
  wgt_ref[0] = p * inv_l

  attn = jax.lax.dot_general(p.astype(jnp.bfloat16), cb,
                             (((1,), (0,)), ((), ())),
                             preferred_element_type=jnp.float32) * inv_l

  w_top = wattn_ref[:, :D].astype(jnp.bfloat16)
  w_bot = wattn_ref[:, D:].astype(jnp.bfloat16)
  h = (jax.lax.dot_general(qb, w_top, (((1,), (1,)), ((), ())),
                           preferred_element_type=jnp.float32) +
       jax.lax.dot_general(attn.astype(jnp.bfloat16), w_bot,
                           (((1,), (1,)), ((), ())),
                           preferred_element_type=jnp.float32))
  out = jnp.tanh(h)
  out_ref[0] = out.astype(out_ref.dtype)

  z = (jax.lax.dot_general(wsw_ref[0:1, :D], q32, (((1,), (1,)), ((), ())),
                           preferred_element_type=jnp.float32) +
       jax.lax.dot_general(wsw_ref[0:1, D:], out, (((1,), (1,)), ((), ())),
                           preferred_element_type=jnp.float32) +
       bsw_ref[...])
  sw_ref[0] = jax.nn.sigmoid(z)


def _round_up(x, m):
  return ((x + m - 1) // m) * m


def kernel(inp, context, atten_mask, w_attn, w_switch, b_switch):
  B, T, D = inp.shape
  _, S, _ = context.shape

  b_sw = jnp.asarray(b_switch, jnp.float32).reshape(1, 1)

  tm = min(512, _round_up(T, 8))
  t_pad, s_pad = _round_up(T, tm), _round_up(S, 128)
  nt = t_pad // tm

  q = jnp.asarray(inp, jnp.float32)
  ctx = jnp.asarray(context, jnp.float32)
  msk = jnp.asarray(atten_mask, jnp.float32)
  if t_pad != T:
    q = jnp.pad(q, ((0, 0), (0, t_pad - T), (0, 0)))
    msk = jnp.pad(msk, ((0, 0), (0, t_pad - T), (0, 0)))
  if s_pad != S:
    ctx = jnp.pad(ctx, ((0, 0), (0, s_pad - S), (0, 0)))
    msk = jnp.pad(msk, ((0, 0), (0, 0), (0, s_pad - S)))

  out_shapes = (
      jax.ShapeDtypeStruct((B, t_pad, D), jnp.float32),
      jax.ShapeDtypeStruct((B, t_pad, s_pad), jnp.float32),
      jax.ShapeDtypeStruct((B, 1, t_pad), jnp.float32),
  )

  import functools
  kfn = functools.partial(_pd_kernel, D=D)

  out, weight, switch = pl.pallas_call(
      kfn,
      out_shape=out_shapes,
      grid=(B, nt),
      in_specs=[
          pl.BlockSpec((1, tm, D), lambda b, i: (b, i, 0)),
          pl.BlockSpec((1, s_pad, D), lambda b, i: (b, 0, 0)),
          pl.BlockSpec((1, tm, s_pad), lambda b, i: (b, i, 0)),
          pl.BlockSpec((D, 2 * D), lambda b, i: (0, 0)),
          pl.BlockSpec((1, 2 * D), lambda b, i: (0, 0)),
          pl.BlockSpec((1, 1), lambda b, i: (0, 0)),
      ],
      out_specs=[
          pl.BlockSpec((1, tm, D), lambda b, i: (b, i, 0)),
          pl.BlockSpec((1, tm, s_pad), lambda b, i: (b, i, 0)),
          pl.BlockSpec((1, 1, tm), lambda b, i: (b, 0, i)),
      ],
      compiler_params=pltpu.CompilerParams(
          dimension_semantics=("parallel", "parallel"),
          vmem_limit_bytes=48 << 20),
  )(q, ctx, msk, jnp.asarray(w_attn, jnp.float32),
    jnp.asarray(w_switch, jnp.float32), b_sw)

  switch = switch.reshape(B, t_pad, 1)
  return out[:, :T, :], weight[:, :T, :S], switch[:, :T, :]


# speedup vs baseline: 2.0500x; 1.0970x over previous
import functools

import jax
import jax.numpy as jnp
from jax.experimental import pallas as pl
from jax.experimental.pallas import tpu as pltpu


def _pd_kernel(q_ref, ctx_ref, msk_ref, wattn_ref, wsw_ref, bsw_ref,
               out_ref, wgt_ref, sw_ref, *, D, bb):
  w_top = wattn_ref[:, :D].astype(jnp.bfloat16)
  w_bot = wattn_ref[:, D:].astype(jnp.bfloat16)

  for j in range(bb):
    q32 = q_ref[j]
    qb = q32.astype(jnp.bfloat16)
    cb = ctx_ref[j].astype(jnp.bfloat16)

    scores = jax.lax.dot_general(qb, cb, (((1,), (1,)), ((), ())),
                                 preferred_element_type=jnp.float32)
    scores = jnp.where(msk_ref[j] != 0.0, scores, jnp.float32(-1e30))

    m = jnp.max(scores, axis=-1, keepdims=True)
    p = jnp.exp(scores - m)
    l = jnp.sum(p, axis=-1, keepdims=True)
    inv_l = pl.reciprocal(l, approx=False)
    wgt_ref[j] = p * inv_l

    attn = jax.lax.dot_general(p.astype(jnp.bfloat16), cb,
                               (((1,), (0,)), ((), ())),
                               preferred_element_type=jnp.float32) * inv_l

    h = (jax.lax.dot_general(qb, w_top, (((1,), (1,)), ((), ())),
                             preferred_element_type=jnp.float32) +
         jax.lax.dot_general(attn.astype(jnp.bfloat16), w_bot,
                             (((1,), (1,)), ((), ())),
                             preferred_element_type=jnp.float32))
    out = jnp.tanh(h)
    out_ref[j] = out.astype(out_ref.dtype)

    z = (jax.lax.dot_general(wsw_ref[0:1, :D], q32, (((1,), (1,)), ((), ())),
                             preferred_element_type=jnp.float32) +
         jax.lax.dot_general(wsw_ref[0:1, D:], out, (((1,), (1,)), ((), ())),
                             preferred_element_type=jnp.float32) +
         bsw_ref[...])
    sw_ref[j] = jax.nn.sigmoid(z)


def _round_up(x, m):
  return ((x + m - 1) // m) * m


def kernel(inp, context, atten_mask, w_attn, w_switch, b_switch):
  B, T, D = inp.shape
  _, S, _ = context.shape

  b_sw = jnp.asarray(b_switch, jnp.float32).reshape(1, 1)

  bb = 2 if B % 2 == 0 else 1
  tm = min(512, _round_up(T, 8))
  t_pad, s_pad = _round_up(T, tm), _round_up(S, 128)
  nt = t_pad // tm

  q = jnp.asarray(inp, jnp.float32)
  ctx = jnp.asarray(context, jnp.float32)
  msk = jnp.asarray(atten_mask, jnp.float32)
  if t_pad != T:
    q = jnp.pad(q, ((0, 0), (0, t_pad - T), (0, 0)))
    msk = jnp.pad(msk, ((0, 0), (0, t_pad - T), (0, 0)))
  if s_pad != S:
    ctx = jnp.pad(ctx, ((0, 0), (0, s_pad - S), (0, 0)))
    msk = jnp.pad(msk, ((0, 0), (0, 0), (0, s_pad - S)))

  out_shapes = (
      jax.ShapeDtypeStruct((B, t_pad, D), jnp.float32),
      jax.ShapeDtypeStruct((B, t_pad, s_pad), jnp.float32),
      jax.ShapeDtypeStruct((B, 1, t_pad), jnp.float32),
  )

  kfn = functools.partial(_pd_kernel, D=D, bb=bb)

  out, weight, switch = pl.pallas_call(
      kfn,
      out_shape=out_shapes,
      grid=(B // bb, nt),
      in_specs=[
          pl.BlockSpec((bb, tm, D), lambda b, i: (b, i, 0)),
          pl.BlockSpec((bb, s_pad, D), lambda b, i: (b, 0, 0)),
          pl.BlockSpec((bb, tm, s_pad), lambda b, i: (b, i, 0)),
          pl.BlockSpec((D, 2 * D), lambda b, i: (0, 0)),
          pl.BlockSpec((1, 2 * D), lambda b, i: (0, 0)),
          pl.BlockSpec((1, 1), lambda b, i: (0, 0)),
      ],
      out_specs=[
          pl.BlockSpec((bb, tm, D), lambda b, i: (b, i, 0)),
          pl.BlockSpec((bb, tm, s_pad), lambda b, i: (b, i, 0)),
          pl.BlockSpec((bb, 1, tm), lambda b, i: (b, 0, i)),
      ],
      compiler_params=pltpu.CompilerParams(
          dimension_semantics=("parallel", "parallel"),
          vmem_limit_bytes=56 << 20),
  )(q, ctx, msk, jnp.asarray(w_attn, jnp.float32),
    jnp.asarray(w_switch, jnp.float32), b_sw)

  switch = switch.reshape(B, t_pad, 1)
  return out[:, :T, :], weight[:, :T, :S], switch[:, :T, :]
